# baseline (speedup 1.0000x reference)
import jax
import jax.numpy as jnp
from jax.experimental import pallas as pl

N = 256
H = 256
A = 64
BI = 32
G = 4
SR = BI // G
NEG = -1e30


def _prep(t_ref, ts_ref, ss_ref, w1_ref, b1_ref, w2_ref, b2_ref,
          v_ref, c_ref, bias_ref):
    tp2 = jax.lax.dot_general(
        t_ref[...], w2_ref[...], (((1,), (1,)), ((), ())),
        preferred_element_type=jnp.float32) + b2_ref[0, :][None, :]
    v_ref[...] = jax.lax.dot_general(
        tp2, w1_ref[...], (((1,), (0,)), ((), ())),
        preferred_element_type=jnp.float32)
    c_ref[...] = jnp.sum(tp2 * b1_ref[0, :][None, :], axis=1,
                         keepdims=True)
    m = jnp.logical_and(ts_ref[0, :] == 1.0,
                        ss_ref[0, :] == 0.0).astype(jnp.float32)
    en = jnp.sum(m)
    scale = en * jax.lax.rsqrt(jnp.float32(A))
    c_ref[...] = c_ref[...] * scale
    v_ref[...] = v_ref[...] * scale
    rows = jax.lax.broadcasted_iota(jnp.int32, (N, N), 0)
    cols = jax.lax.broadcasted_iota(jnp.int32, (N, N), 1)
    allowed = ((rows != cols) & (m[:, None] > 0.0) & (m[None, :] > 0.0))
    bias_ref[...] = jnp.where(allowed, 0.0, NEG)


def _attn_block(s_ref, v_ref, c_ref, bias_ref, out_ref):
    eye3 = (jax.lax.broadcasted_iota(jnp.int32, (SR, 1, SR), 0) ==
            jax.lax.broadcasted_iota(jnp.int32, (SR, 1, SR), 2)
            ).astype(jnp.float32)
    for g in range(G):
        lo = g * SR
        s3 = s_ref[0, lo:lo + SR]
        s2 = s3.reshape(SR * N, H)
        vblk = v_ref[lo:lo + SR, :]
        p = jax.lax.dot_general(
            s2, vblk, (((1,), (1,)), ((), ())),
            preferred_element_type=jnp.float32)
        p3 = p.reshape(SR, N, SR)
        sm = jnp.sum(p3 * eye3, axis=2)
        logits = sm + c_ref[lo:lo + SR, :] + bias_ref[lo:lo + SR, :]
        num = jnp.exp(logits)
        den = jnp.sum(num, axis=1, keepdims=True)
        inv = 1.0 / jnp.where(den == 0.0, 1.0, den)
        score = num * inv
        scm = (score[:, :, None] * eye3).reshape(SR * N, SR)
        out_ref[lo:lo + SR, :] = jax.lax.dot_general(
            scm, s2, (((0,), (0,)), ((), ())),
            preferred_element_type=jnp.float32)


@jax.jit
def _edge_to_node_attention(spatial_ht_list, temporal_ht_list, ts_mask,
                            same_scene_mask, W1_w, W1_b, W2_w, W2_b):
    v, c, bias = pl.pallas_call(
        _prep,
        out_shape=(
            jax.ShapeDtypeStruct((N, H), jnp.float32),
            jax.ShapeDtypeStruct((N, 1), jnp.float32),
            jax.ShapeDtypeStruct((N, N), jnp.float32),
        ),
    )(temporal_ht_list, ts_mask, same_scene_mask, W1_w, W1_b, W2_w, W2_b)

    return pl.pallas_call(
        _attn_block,
        grid=(N // BI,),
        in_specs=[
            pl.BlockSpec((1, BI, N, H), lambda i: (0, i, 0, 0)),
            pl.BlockSpec((BI, H), lambda i: (i, 0)),
            pl.BlockSpec((BI, 1), lambda i: (i, 0)),
            pl.BlockSpec((BI, N), lambda i: (i, 0)),
        ],
        out_specs=pl.BlockSpec((BI, H), lambda i: (i, 0)),
        out_shape=jax.ShapeDtypeStruct((N, H), jnp.float32),
    )(spatial_ht_list, v, c, bias)


def kernel(spatial_ht_list, temporal_ht_list, ts_mask, same_scene_mask,
           W1_w, W1_b, W2_w, W2_b):
    return _edge_to_node_attention(
        spatial_ht_list, temporal_ht_list,
        ts_mask.reshape(1, N), same_scene_mask.reshape(1, N),
        W1_w, W1_b.reshape(1, A), W2_w, W2_b.reshape(1, A))


# speedup vs baseline: 3.1165x; 3.1165x over previous
import jax
import jax.numpy as jnp
from jax.experimental import pallas as pl

N = 256
H = 256
A = 64
BI = 16


def _prep(t_ref, ts_ref, ss_ref, w1_ref, b1_ref, w2_ref, b2_ref,
          v_ref, c_ref, bias_ref):
    tp2 = jax.lax.dot_general(
        t_ref[...], w2_ref[...], (((1,), (1,)), ((), ())),
        preferred_element_type=jnp.float32) + b2_ref[0, :][None, :]
    m = jnp.logical_and(ts_ref[0, :] == 1.0,
                        ss_ref[0, :] == 0.0).astype(jnp.float32)
    en = jnp.sum(m)
    scale = en * jax.lax.rsqrt(jnp.float32(A))
    v_ref[...] = jax.lax.dot_general(
        tp2, w1_ref[...], (((1,), (0,)), ((), ())),
        preferred_element_type=jnp.float32) * scale
    c_ref[...] = jnp.sum(tp2 * b1_ref[0, :][None, :], axis=1,
                         keepdims=True) * scale
    rows = jax.lax.broadcasted_iota(jnp.int32, (N, N), 0)
    cols = jax.lax.broadcasted_iota(jnp.int32, (N, N), 1)
    allowed = ((rows != cols) & (m[:, None] > 0.0) & (m[None, :] > 0.0))
    bias_ref[...] = jnp.where(allowed, 0.0, -1e30)


def _attn_block(s_ref, v_ref, c_ref, bias_ref, out_ref):
    s3 = s_ref[0]
    s2 = s3.reshape(BI * N, H)
    p = jax.lax.dot_general(
        s2, v_ref[...], (((1,), (1,)), ((), ())),
        preferred_element_type=jnp.float32)
    p3 = p.reshape(BI, N, BI)
    eye = (jax.lax.broadcasted_iota(jnp.int32, (BI, 1, BI), 0) ==
           jax.lax.broadcasted_iota(jnp.int32, (BI, 1, BI), 2)
           ).astype(jnp.float32)
    sm = jnp.sum(p3 * eye, axis=2)
    logits = sm + c_ref[...] + bias_ref[...]
    num = jnp.exp(logits)
    den = jnp.sum(num, axis=1, keepdims=True)
    inv = 1.0 / jnp.where(den == 0.0, 1.0, den)
    score = num * inv
    out_ref[...] = jnp.sum(s3 * score[:, :, None], axis=1)


@jax.jit
def _edge_to_node_attention(spatial_ht_list, temporal_ht_list, ts_mask,
                            same_scene_mask, W1_w, W1_b, W2_w, W2_b):
    v, c, bias = pl.pallas_call(
        _prep,
        out_shape=(
            jax.ShapeDtypeStruct((N, H), jnp.float32),
            jax.ShapeDtypeStruct((N, 1), jnp.float32),
            jax.ShapeDtypeStruct((N, N), jnp.float32),
        ),
    )(temporal_ht_list, ts_mask, same_scene_mask, W1_w, W1_b, W2_w, W2_b)

    return pl.pallas_call(
        _attn_block,
        grid=(N // BI,),
        in_specs=[
            pl.BlockSpec((1, BI, N, H), lambda i: (0, i, 0, 0)),
            pl.BlockSpec((BI, H), lambda i: (i, 0)),
            pl.BlockSpec((BI, 1), lambda i: (i, 0)),
            pl.BlockSpec((BI, N), lambda i: (i, 0)),
        ],
        out_specs=pl.BlockSpec((BI, H), lambda i: (i, 0)),
        out_shape=jax.ShapeDtypeStruct((N, H), jnp.float32),
    )(spatial_ht_list, v, c, bias)


def kernel(spatial_ht_list, temporal_ht_list, ts_mask, same_scene_mask,
           W1_w, W1_b, W2_w, W2_b):
    return _edge_to_node_attention(
        spatial_ht_list, temporal_ht_list,
        ts_mask.reshape(1, N), same_scene_mask.reshape(1, N),
        W1_w, W1_b.reshape(1, A), W2_w, W2_b.reshape(1, A))
